# baseline (speedup 1.0000x reference)
import jax
import jax.numpy as jnp
import numpy as np
from jax.experimental import pallas as pl
from jax.experimental.pallas import tpu as pltpu

_D = 480
_NB = 128
_NBLK = 4
_A = 20
_M = 500
_B = 2
_P = _B * _A * _A
_N = _B * _A
_CUT = 5.0
_WD = _CUT / _NB
_INV_SQ_DEG = np.float32(1.0 / np.sqrt(15.0))
_INV_SQ_NODES = np.float32(1.0 / np.sqrt(20.0))
_C3 = np.float32(np.sqrt(3.0))
_C5 = np.float32(np.sqrt(5.0))
_C15 = np.float32(np.sqrt(15.0))
_CENTERS_NP = np.linspace(0.0, _CUT, _NB).astype(np.float32)


def _build_shmap():
    m = np.zeros((9, 480), np.float32)
    m[0, :128] = 1.0
    for t in range(64):
        for k in range(3):
            m[1 + k, 128 + 3 * t + k] = 1.0
    for t in range(32):
        for k in range(5):
            m[4 + k, 320 + 5 * t + k] = 1.0
    return m


def _build_sel():
    p = np.arange(_P)
    r = np.arange(_N)[:, None]
    s1 = (p[None, :] // _A == r).astype(np.float32)
    s2 = ((p[None, :] // (_A * _A)) * _A + p[None, :] % _A == r
          ).astype(np.float32)
    smol = (np.arange(_B)[:, None] == np.arange(_N)[None, :] // _A
            ).astype(np.float32)
    return s1, s2, smol


_SHMAP_NP = _build_shmap()
_S1_NP, _S2_NP, _SMOL_NP = _build_sel()


def _silu(z):
    return z * jax.nn.sigmoid(z)


def _dsilu(z):
    s = jax.nn.sigmoid(z)
    return s * (1.0 + z * (1.0 - s))


def _dot(a, b):
    return jnp.dot(a, b, preferred_element_type=jnp.float32)


def _mlp_fwd(rbf, W1, b1, W2, b2, W3):
    z1 = _dot(rbf, W1) + b1
    f = _silu(z1)
    z2 = _dot(f, W2) + b2
    g = _silu(z2)
    w = _dot(g, W3)
    return z1, z2, w


def _mlp_bwd(w_bar, z1, z2, W1, W2, W3):
    g_bar = _dot(w_bar, W3.T)
    z2_bar = g_bar * _dsilu(z2)
    f_bar = _dot(z2_bar, W2.T)
    z1_bar = f_bar * _dsilu(z1)
    return _dot(z1_bar, W1.T)


def _block_compute(Rf, oh, maskP, shmap, centers, S1, S1T, S2, S2T, Smol,
                   atom_table, deg_W1, deg_b1, deg_W2, deg_b2, deg_W3,
                   blk_W1, blk_b1, blk_W2, blk_b2, blk_W3, blk_Wv, blk_Wo,
                   ln_g, ln_b, head_W1, head_W2):
    Pj = _dot(S2T, Rf)
    Pi = _dot(S1T, Rf)
    ev = Pj - Pi
    evx = ev[:, 0:1]
    evy = ev[:, 1:2]
    evz = ev[:, 2:3]
    s = evx * evx + evy * evy + evz * evz + 1e-12
    el = jnp.sqrt(s)
    inv = 1.0 / el
    ux = evx * inv
    uy = evy * inv
    uz = evz * inv

    sh9m = jnp.concatenate([
        jnp.ones_like(ux),
        _C3 * ux, _C3 * uy, _C3 * uz,
        _C15 * ux * uy, _C15 * uy * uz,
        (_C5 * 0.5) * (2.0 * uz * uz - ux * ux - uy * uy),
        _C15 * ux * uz,
        (_C15 * 0.5) * (ux * ux - uy * uy),
    ], axis=1) * maskP
    shfM = _dot(sh9m, shmap)
    rbf = jnp.exp(-0.5 * jnp.square((el - centers) / _WD))

    x = _dot(oh, atom_table)
    z1_0, z2_0, w0 = _mlp_fwd(rbf, deg_W1, deg_b1, deg_W2, deg_b2, deg_W3)
    x = x + _dot(S1, w0 * shfM) * _INV_SQ_DEG

    xs = []
    mlp_cache = []
    for b in range(_NBLK):
        xs.append(x)
        z1, z2, w = _mlp_fwd(rbf, blk_W1[b], blk_b1[b], blk_W2[b], blk_b2[b],
                             blk_W3[b])
        mlp_cache.append((z1, z2, w))
        v = _dot(x, blk_Wv[b])
        vP = _dot(S2T, v)
        agg = _dot(S1, (w * shfM) * vP)
        x = x + _dot(agg * _INV_SQ_DEG, blk_Wo[b])
    x5 = x

    mu = jnp.mean(x5, axis=-1, keepdims=True)
    var = jnp.mean(jnp.square(x5 - mu), axis=-1, keepdims=True)
    istd = 1.0 / jnp.sqrt(var + 1e-5)
    y = (x5 - mu) * istd
    xn = y * ln_g + ln_b
    h = _dot(xn, head_W1)
    hp = jnp.concatenate([_silu(h[:, :128]), h[:, 128:]], axis=1)
    node_e = _dot(hp, head_W2)
    energy = _dot(Smol, node_e) * _INV_SQ_NODES

    hp_bar = jnp.broadcast_to((head_W2.T * _INV_SQ_NODES), (x5.shape[0], _D))
    h_bar = jnp.concatenate(
        [hp_bar[:, :128] * _dsilu(h[:, :128]), hp_bar[:, 128:]], axis=1)
    xn_bar = _dot(h_bar, head_W1.T)
    y_bar = xn_bar * ln_g
    x_bar = istd * (y_bar
                    - jnp.mean(y_bar, axis=-1, keepdims=True)
                    - y * jnp.mean(y_bar * y, axis=-1, keepdims=True))

    rbf_bar = jnp.zeros(rbf.shape, jnp.float32)
    sh9m_bar = jnp.zeros(sh9m.shape, jnp.float32)

    for b in range(_NBLK - 1, -1, -1):
        z1, z2, w = mlp_cache[b]
        v = _dot(xs[b], blk_Wv[b])
        vP = _dot(S2T, v)
        agg_bar = _dot(x_bar, blk_Wo[b].T) * _INV_SQ_DEG
        aggbarP = _dot(S1T, agg_bar)
        T_bar = aggbarP * vP
        v_bar = _dot(S2, (w * shfM) * aggbarP)
        w_bar = T_bar * shfM
        sh9m_bar = sh9m_bar + _dot(T_bar * w, shmap.T)
        rbf_bar = rbf_bar + _mlp_bwd(w_bar, z1, z2, blk_W1[b], blk_W2[b],
                                     blk_W3[b])
        x_bar = x_bar + _dot(v_bar, blk_Wv[b].T)

    T0_bar = _dot(S1T, x_bar) * _INV_SQ_DEG
    w_bar = T0_bar * shfM
    sh9m_bar = sh9m_bar + _dot(T0_bar * w0, shmap.T)
    rbf_bar = rbf_bar + _mlp_bwd(w_bar, z1_0, z2_0, deg_W1, deg_W2, deg_W3)

    el_bar = ((rbf_bar * rbf * (centers - el)).sum(axis=-1, keepdims=True)
              * (1.0 / (_WD * _WD)))

    shb = sh9m_bar * maskP
    s1c = shb[:, 1:2]; s2c = shb[:, 2:3]; s3c = shb[:, 3:4]
    s4c = shb[:, 4:5]; s5c = shb[:, 5:6]; s6c = shb[:, 6:7]
    s7c = shb[:, 7:8]; s8c = shb[:, 8:9]
    ux_bar = _C3 * s1c + _C15 * (uy * s4c + uz * s7c) - _C5 * ux * s6c + _C15 * ux * s8c
    uy_bar = _C3 * s2c + _C15 * (ux * s4c + uz * s5c) - _C5 * uy * s6c - _C15 * uy * s8c
    uz_bar = _C3 * s3c + _C15 * (uy * s5c + ux * s7c) + 2.0 * _C5 * uz * s6c

    inv_bar = ux_bar * evx + uy_bar * evy + uz_bar * evz
    el_bar = el_bar - inv_bar * inv * inv
    s_bar = el_bar * 0.5 * inv
    evb = jnp.concatenate([
        ux_bar * inv + 2.0 * evx * s_bar,
        uy_bar * inv + 2.0 * evy * s_bar,
        uz_bar * inv + 2.0 * evz * s_bar,
    ], axis=1)

    F = _dot(S1, evb) - _dot(S2, evb)
    return energy, F


def _pallas_body(r_ref, oh_ref, mask_ref, shmap_ref, cen_ref,
                 s1_ref, s1t_ref, s2_ref, s2t_ref, smol_ref,
                 at_ref, dW1_ref, db1_ref, dW2_ref, db2_ref, dW3_ref,
                 bW1_ref, bb1_ref, bW2_ref, bb2_ref, bW3_ref, bWv_ref, bWo_ref,
                 lng_ref, lnb_ref, hW1_ref, hW2_ref,
                 e_ref, f_ref):
    energy, F = _block_compute(
        r_ref[...].reshape(_N, 3), oh_ref[...].reshape(_N, 5),
        mask_ref[...].reshape(_P, 1),
        shmap_ref[...], cen_ref[...],
        s1_ref[...], s1t_ref[...], s2_ref[...], s2t_ref[...], smol_ref[...],
        at_ref[...], dW1_ref[...], db1_ref[...], dW2_ref[...], db2_ref[...],
        dW3_ref[...],
        bW1_ref[...], bb1_ref[...], bW2_ref[...], bb2_ref[...], bW3_ref[...],
        bWv_ref[...], bWo_ref[...],
        lng_ref[...], lnb_ref[...], hW1_ref[...], hW2_ref[...])
    e_ref[:, 0, :] = energy
    f_ref[...] = F.reshape(_B, _A, 3)


def kernel(node_atom, pos, batch, edge_src, edge_dst, atom_table,
           deg_W1, deg_b1, deg_W2, deg_b2, deg_W3,
           blk_W1, blk_b1, blk_W2, blk_b2, blk_W3, blk_Wv, blk_Wo,
           ln_g, ln_b, head_W1, head_W2):
    node_atom = node_atom.astype(jnp.int32)
    edge_src = edge_src.astype(jnp.int32)
    edge_dst = edge_dst.astype(jnp.int32)

    R = pos.reshape(_M, _A, 3)
    onehot = (node_atom[:, None] ==
              jnp.array([1, 6, 7, 8, 9], jnp.int32)[None, :]
              ).astype(jnp.float32).reshape(_M, _A, 5)
    maskP = jnp.zeros((_M, _A * _A, 1), jnp.float32).at[
        edge_dst // _A, (edge_dst % _A) * _A + edge_src % _A, 0].set(1.0)

    grid = (_M // _B,)
    full = lambda shape: pl.BlockSpec(shape, lambda m: tuple(0 for _ in shape))
    in_specs = [
        pl.BlockSpec((_B, _A, 3), lambda m: (m, 0, 0)),
        pl.BlockSpec((_B, _A, 5), lambda m: (m, 0, 0)),
        pl.BlockSpec((_B, _A * _A, 1), lambda m: (m, 0, 0)),
        full((9, _D)),
        full((1, _NB)),
        full((_N, _P)), full((_P, _N)),
        full((_N, _P)), full((_P, _N)),
        full((_B, _N)),
        full((5, _D)),
        full((_NB, 64)), full((1, 64)), full((64, 64)), full((1, 64)),
        full((64, _D)),
        full((_NBLK, _NB, 64)), full((_NBLK, 1, 64)),
        full((_NBLK, 64, 64)), full((_NBLK, 1, 64)),
        full((_NBLK, 64, _D)),
        full((_NBLK, _D, _D)), full((_NBLK, _D, _D)),
        full((1, _D)), full((1, _D)),
        full((_D, _D)), full((_D, 1)),
    ]
    out_specs = [
        pl.BlockSpec((_B, 1, 1), lambda m: (m, 0, 0)),
        pl.BlockSpec((_B, _A, 3), lambda m: (m, 0, 0)),
    ]
    energy, fT = pl.pallas_call(
        _pallas_body,
        grid=grid,
        in_specs=in_specs,
        out_specs=out_specs,
        out_shape=[
            jax.ShapeDtypeStruct((_M, 1, 1), jnp.float32),
            jax.ShapeDtypeStruct((_M, _A, 3), jnp.float32),
        ],
        compiler_params=pltpu.CompilerParams(
            dimension_semantics=("arbitrary",)),
    )(R, onehot, maskP, jnp.asarray(_SHMAP_NP),
      jnp.asarray(_CENTERS_NP).reshape(1, _NB),
      jnp.asarray(_S1_NP), jnp.asarray(_S1_NP.T),
      jnp.asarray(_S2_NP), jnp.asarray(_S2_NP.T),
      jnp.asarray(_SMOL_NP),
      atom_table,
      deg_W1, deg_b1.reshape(1, 64), deg_W2, deg_b2.reshape(1, 64), deg_W3,
      blk_W1, blk_b1.reshape(_NBLK, 1, 64), blk_W2, blk_b2.reshape(_NBLK, 1, 64),
      blk_W3, blk_Wv, blk_Wo,
      ln_g.reshape(1, _D), ln_b.reshape(1, _D), head_W1, head_W2)

    forces = fT.reshape(_M * _A, 3)
    return (energy.reshape(_M, 1), forces)


# speedup vs baseline: 1.0192x; 1.0192x over previous
import jax
import jax.numpy as jnp
import numpy as np
from jax.experimental import pallas as pl
from jax.experimental.pallas import tpu as pltpu

_D = 480
_NB = 128
_NBLK = 4
_A = 20
_M = 500
_CUT = 5.0
_WD = _CUT / _NB
_INV_SQ_DEG = np.float32(1.0 / np.sqrt(15.0))
_INV_SQ_NODES = np.float32(1.0 / np.sqrt(20.0))
_C3 = np.float32(np.sqrt(3.0))
_C5 = np.float32(np.sqrt(5.0))
_C15 = np.float32(np.sqrt(15.0))
_CENTERS_NP = np.linspace(0.0, _CUT, _NB).astype(np.float32)


def _build_shmap():
    m = np.zeros((9, 480), np.float32)
    m[0, :128] = 1.0
    for t in range(64):
        for k in range(3):
            m[1 + k, 128 + 3 * t + k] = 1.0
    for t in range(32):
        for k in range(5):
            m[4 + k, 320 + 5 * t + k] = 1.0
    return m


_SHMAP_NP = _build_shmap()


def _silu(z):
    return z * jax.nn.sigmoid(z)


def _dsilu(z):
    s = jax.nn.sigmoid(z)
    return s * (1.0 + z * (1.0 - s))


def _mlp_fwd(rbf, W1, b1, W2, b2, W3):
    z1 = jnp.dot(rbf, W1, preferred_element_type=jnp.float32) + b1
    f = _silu(z1)
    z2 = jnp.dot(f, W2, preferred_element_type=jnp.float32) + b2
    g = _silu(z2)
    w = jnp.dot(g, W3, preferred_element_type=jnp.float32)
    return z1, z2, w


def _mlp_bwd(w_bar, z1, z2, W1, W2, W3):
    g_bar = jnp.dot(w_bar, W3.T, preferred_element_type=jnp.float32)
    z2_bar = g_bar * _dsilu(z2)
    f_bar = jnp.dot(z2_bar, W2.T, preferred_element_type=jnp.float32)
    z1_bar = f_bar * _dsilu(z1)
    return jnp.dot(z1_bar, W1.T, preferred_element_type=jnp.float32)


def _block_compute(Rx, Ry, Rz, onehot, mask3, shmap, centers,
                   atom_table, deg_W1, deg_b1, deg_W2, deg_b2, deg_W3,
                   blk_W1, blk_b1, blk_W2, blk_b2, blk_W3, blk_Wv, blk_Wo,
                   ln_g, ln_b, head_W1, head_W2t):
    B = Rx.shape[0]
    P = B * _A * _A
    N = B * _A

    evx = Rx[:, None, :] - Rx[:, :, None]
    evy = Ry[:, None, :] - Ry[:, :, None]
    evz = Rz[:, None, :] - Rz[:, :, None]
    s = evx * evx + evy * evy + evz * evz + 1e-12
    el = jnp.sqrt(s)
    inv = 1.0 / el
    ux = evx * inv
    uy = evy * inv
    uz = evz * inv

    sh_comps = [
        jnp.ones_like(ux),
        _C3 * ux, _C3 * uy, _C3 * uz,
        _C15 * ux * uy, _C15 * uy * uz,
        (_C5 * 0.5) * (2.0 * uz * uz - ux * ux - uy * uy),
        _C15 * ux * uz,
        (_C15 * 0.5) * (ux * ux - uy * uy),
    ]
    sh9m = jnp.stack(sh_comps, axis=-1) * mask3[..., None]
    sh9m_flat = sh9m.reshape(P, 9)
    shfM = jnp.dot(sh9m_flat, shmap, preferred_element_type=jnp.float32)

    rbf = jnp.exp(-0.5 * jnp.square((el[..., None] - centers) / _WD)
                  ).reshape(P, _NB)

    x = jnp.dot(onehot.reshape(N, 5), atom_table,
                preferred_element_type=jnp.float32)

    z1_0, z2_0, w0 = _mlp_fwd(rbf, deg_W1, deg_b1, deg_W2, deg_b2, deg_W3)
    T0 = (w0 * shfM).reshape(B, _A, _A, _D)
    x = x + T0.sum(axis=2).reshape(N, _D) * _INV_SQ_DEG

    xs = []
    mlp_cache = []
    for b in range(_NBLK):
        xs.append(x)
        z1, z2, w = _mlp_fwd(rbf, blk_W1[b], blk_b1[b], blk_W2[b], blk_b2[b],
                             blk_W3[b])
        mlp_cache.append((z1, z2, w))
        v = jnp.dot(x, blk_Wv[b], preferred_element_type=jnp.float32)
        T4 = (w * shfM).reshape(B, _A, _A, _D)
        agg = (T4 * v.reshape(B, 1, _A, _D)).sum(axis=2).reshape(N, _D)
        x = x + jnp.dot(agg * _INV_SQ_DEG, blk_Wo[b],
                        preferred_element_type=jnp.float32)
    x5 = x

    mu = jnp.mean(x5, axis=-1, keepdims=True)
    var = jnp.mean(jnp.square(x5 - mu), axis=-1, keepdims=True)
    istd = 1.0 / jnp.sqrt(var + 1e-5)
    y = (x5 - mu) * istd
    xn = y * ln_g + ln_b
    h = jnp.dot(xn, head_W1, preferred_element_type=jnp.float32)
    hp = jnp.concatenate([_silu(h[:, :128]), h[:, 128:]], axis=1)
    node_e = jnp.sum(hp * head_W2t, axis=-1)
    energy = node_e.reshape(B, _A).sum(axis=1, keepdims=True) * _INV_SQ_NODES

    hp_bar = jnp.broadcast_to(head_W2t * _INV_SQ_NODES, (N, _D))
    h_bar = jnp.concatenate(
        [hp_bar[:, :128] * _dsilu(h[:, :128]), hp_bar[:, 128:]], axis=1)
    xn_bar = jnp.dot(h_bar, head_W1.T, preferred_element_type=jnp.float32)
    y_bar = xn_bar * ln_g
    x_bar = istd * (y_bar
                    - jnp.mean(y_bar, axis=-1, keepdims=True)
                    - y * jnp.mean(y_bar * y, axis=-1, keepdims=True))

    rbf_bar = jnp.zeros((P, _NB), jnp.float32)
    sh9m_bar = jnp.zeros((P, 9), jnp.float32)

    for b in range(_NBLK - 1, -1, -1):
        z1, z2, w = mlp_cache[b]
        v = jnp.dot(xs[b], blk_Wv[b], preferred_element_type=jnp.float32)
        agg_bar = jnp.dot(x_bar, blk_Wo[b].T,
                          preferred_element_type=jnp.float32) * _INV_SQ_DEG
        agg_bar4 = agg_bar.reshape(B, _A, 1, _D)
        T4 = (w * shfM).reshape(B, _A, _A, _D)
        v_bar = (T4 * agg_bar4).sum(axis=1).reshape(N, _D)
        T_bar = (agg_bar4 * v.reshape(B, 1, _A, _D)).reshape(P, _D)
        w_bar = T_bar * shfM
        Tw = T_bar * w
        sh9m_bar = sh9m_bar + jnp.dot(Tw, shmap.T, preferred_element_type=jnp.float32)
        rbf_bar = rbf_bar + _mlp_bwd(w_bar, z1, z2, blk_W1[b], blk_W2[b], blk_W3[b])
        x_bar = x_bar + jnp.dot(v_bar, blk_Wv[b].T, preferred_element_type=jnp.float32)

    T0_bar = jnp.broadcast_to(
        (x_bar * _INV_SQ_DEG).reshape(B, _A, 1, _D), (B, _A, _A, _D)).reshape(P, _D)
    w_bar = T0_bar * shfM
    Tw = T0_bar * w0
    sh9m_bar = sh9m_bar + jnp.dot(Tw, shmap.T, preferred_element_type=jnp.float32)
    rbf_bar = rbf_bar + _mlp_bwd(w_bar, z1_0, z2_0, deg_W1, deg_W2, deg_W3)

    rbf_bar4 = rbf_bar.reshape(B, _A, _A, _NB)
    rbf4 = rbf.reshape(B, _A, _A, _NB)
    el_bar = ((rbf_bar4 * rbf4 * (centers - el[..., None])).sum(axis=-1)
              * (1.0 / (_WD * _WD)))

    shb = sh9m_bar.reshape(B, _A, _A, 9) * mask3[..., None]
    s1 = shb[..., 1]; s2 = shb[..., 2]; s3 = shb[..., 3]
    s4 = shb[..., 4]; s5 = shb[..., 5]; s6 = shb[..., 6]
    s7 = shb[..., 7]; s8 = shb[..., 8]
    ux_bar = _C3 * s1 + _C15 * (uy * s4 + uz * s7) - _C5 * ux * s6 + _C15 * ux * s8
    uy_bar = _C3 * s2 + _C15 * (ux * s4 + uz * s5) - _C5 * uy * s6 - _C15 * uy * s8
    uz_bar = _C3 * s3 + _C15 * (uy * s5 + ux * s7) + 2.0 * _C5 * uz * s6

    evx_bar = ux_bar * inv
    evy_bar = uy_bar * inv
    evz_bar = uz_bar * inv
    inv_bar = ux_bar * evx + uy_bar * evy + uz_bar * evz
    el_bar = el_bar - inv_bar * inv * inv
    s_bar = el_bar * 0.5 * inv
    evx_bar = evx_bar + 2.0 * evx * s_bar
    evy_bar = evy_bar + 2.0 * evy * s_bar
    evz_bar = evz_bar + 2.0 * evz * s_bar

    Fx = evx_bar.sum(axis=2) - evx_bar.sum(axis=1)
    Fy = evy_bar.sum(axis=2) - evy_bar.sum(axis=1)
    Fz = evz_bar.sum(axis=2) - evz_bar.sum(axis=1)
    return energy, Fx, Fy, Fz


def _pallas_body(rt_ref, oh_ref, mask_ref, shmap_ref, cen_ref,
                 at_ref, dW1_ref, db1_ref, dW2_ref, db2_ref, dW3_ref,
                 bW1_ref, bb1_ref, bW2_ref, bb2_ref, bW3_ref, bWv_ref, bWo_ref,
                 lng_ref, lnb_ref, hW1_ref, hW2_ref,
                 e_ref, f_ref):
    energy, Fx, Fy, Fz = _block_compute(
        rt_ref[:, 0, :], rt_ref[:, 1, :], rt_ref[:, 2, :], oh_ref[...],
        mask_ref[...], shmap_ref[...], cen_ref[0],
        at_ref[...], dW1_ref[...], db1_ref[...], dW2_ref[...], db2_ref[...],
        dW3_ref[...],
        bW1_ref[...], bb1_ref[...], bW2_ref[...], bb2_ref[...], bW3_ref[...],
        bWv_ref[...], bWo_ref[...],
        lng_ref[...], lnb_ref[...], hW1_ref[...], hW2_ref[...])
    e_ref[:, 0, :] = energy
    f_ref[:, 0, :] = Fx
    f_ref[:, 1, :] = Fy
    f_ref[:, 2, :] = Fz


def kernel(node_atom, pos, batch, edge_src, edge_dst, atom_table,
           deg_W1, deg_b1, deg_W2, deg_b2, deg_W3,
           blk_W1, blk_b1, blk_W2, blk_b2, blk_W3, blk_Wv, blk_Wo,
           ln_g, ln_b, head_W1, head_W2):
    B = 2
    node_atom = node_atom.astype(jnp.int32)
    edge_src = edge_src.astype(jnp.int32)
    edge_dst = edge_dst.astype(jnp.int32)

    R3 = pos.reshape(_M, _A, 3).transpose(0, 2, 1)
    onehot = (node_atom[:, None] ==
              jnp.array([1, 6, 7, 8, 9], jnp.int32)[None, :]
              ).astype(jnp.float32).reshape(_M, _A, 5)
    mask = jnp.zeros((_M * _A * _A,), jnp.float32).at[
        edge_dst * _A + edge_src % _A].add(1.0).reshape(_M, _A, _A)
    shmap = jnp.asarray(_SHMAP_NP)

    grid = (_M // B,)
    full = lambda shape: pl.BlockSpec(shape, lambda m: tuple(0 for _ in shape))
    in_specs = [
        pl.BlockSpec((B, 3, _A), lambda m: (m, 0, 0)),
        pl.BlockSpec((B, _A, 5), lambda m: (m, 0, 0)),
        pl.BlockSpec((B, _A, _A), lambda m: (m, 0, 0)),
        full((9, _D)),
        full((1, _NB)),
        full((5, _D)),
        full((_NB, 64)), full((1, 64)), full((64, 64)), full((1, 64)),
        full((64, _D)),
        full((_NBLK, _NB, 64)), full((_NBLK, 1, 64)),
        full((_NBLK, 64, 64)), full((_NBLK, 1, 64)),
        full((_NBLK, 64, _D)),
        full((_NBLK, _D, _D)), full((_NBLK, _D, _D)),
        full((1, _D)), full((1, _D)),
        full((_D, _D)), full((1, _D)),
    ]
    out_specs = [
        pl.BlockSpec((B, 1, 1), lambda m: (m, 0, 0)),
        pl.BlockSpec((B, 3, _A), lambda m: (m, 0, 0)),
    ]
    energy, fT = pl.pallas_call(
        _pallas_body,
        grid=grid,
        in_specs=in_specs,
        out_specs=out_specs,
        out_shape=[
            jax.ShapeDtypeStruct((_M, 1, 1), jnp.float32),
            jax.ShapeDtypeStruct((_M, 3, _A), jnp.float32),
        ],
        compiler_params=pltpu.CompilerParams(
            dimension_semantics=("arbitrary",)),
    )(R3, onehot, mask, shmap, jnp.asarray(_CENTERS_NP).reshape(1, _NB),
      atom_table,
      deg_W1, deg_b1.reshape(1, 64), deg_W2, deg_b2.reshape(1, 64), deg_W3,
      blk_W1, blk_b1.reshape(_NBLK, 1, 64), blk_W2, blk_b2.reshape(_NBLK, 1, 64),
      blk_W3, blk_Wv, blk_Wo,
      ln_g.reshape(1, _D), ln_b.reshape(1, _D), head_W1, head_W2.T)

    forces = fT.transpose(0, 2, 1).reshape(_M * _A, 3)
    return (energy.reshape(_M, 1), forces)


# speedup vs baseline: 1.0726x; 1.0524x over previous
import jax
import jax.numpy as jnp
import numpy as np
from jax.experimental import pallas as pl
from jax.experimental.pallas import tpu as pltpu

_D = 480
_NB = 128
_NBLK = 4
_A = 20
_M = 500
_CUT = 5.0
_WD = _CUT / _NB
_INV_SQ_DEG = np.float32(1.0 / np.sqrt(15.0))
_INV_SQ_NODES = np.float32(1.0 / np.sqrt(20.0))
_C3 = np.float32(np.sqrt(3.0))
_C5 = np.float32(np.sqrt(5.0))
_C15 = np.float32(np.sqrt(15.0))
_CENTERS_NP = np.linspace(0.0, _CUT, _NB).astype(np.float32)


def _build_shmap():
    m = np.zeros((9, 480), np.float32)
    m[0, :128] = 1.0
    for t in range(64):
        for k in range(3):
            m[1 + k, 128 + 3 * t + k] = 1.0
    for t in range(32):
        for k in range(5):
            m[4 + k, 320 + 5 * t + k] = 1.0
    return m


_SHMAP_NP = _build_shmap()


def _silu(z):
    return z * jax.nn.sigmoid(z)


def _dsilu(z):
    s = jax.nn.sigmoid(z)
    return s * (1.0 + z * (1.0 - s))


def _mlp_fwd(rbf, W1, b1, W2, b2, W3):
    z1 = jnp.dot(rbf, W1, preferred_element_type=jnp.float32) + b1
    f = _silu(z1)
    z2 = jnp.dot(f, W2, preferred_element_type=jnp.float32) + b2
    g = _silu(z2)
    w = jnp.dot(g, W3, preferred_element_type=jnp.float32)
    return z1, z2, w


def _mlp_bwd(w_bar, z1, z2, W1, W2, W3):
    g_bar = jnp.dot(w_bar, W3.T, preferred_element_type=jnp.float32)
    z2_bar = g_bar * _dsilu(z2)
    f_bar = jnp.dot(z2_bar, W2.T, preferred_element_type=jnp.float32)
    z1_bar = f_bar * _dsilu(z1)
    return jnp.dot(z1_bar, W1.T, preferred_element_type=jnp.float32)


def _block_compute(Rx, Ry, Rz, onehot, mask3, shmap, centers,
                   atom_table, deg_W1, deg_b1, deg_W2, deg_b2, deg_W3,
                   blk_W1, blk_b1, blk_W2, blk_b2, blk_W3, blk_Wv, blk_Wo,
                   ln_g, ln_b, head_W1, head_W2t):
    B = Rx.shape[0]
    P = B * _A * _A
    N = B * _A

    evx = Rx[:, None, :] - Rx[:, :, None]
    evy = Ry[:, None, :] - Ry[:, :, None]
    evz = Rz[:, None, :] - Rz[:, :, None]
    s = evx * evx + evy * evy + evz * evz + 1e-12
    el = jnp.sqrt(s)
    inv = 1.0 / el
    ux = evx * inv
    uy = evy * inv
    uz = evz * inv

    sh_comps = [
        jnp.ones_like(ux),
        _C3 * ux, _C3 * uy, _C3 * uz,
        _C15 * ux * uy, _C15 * uy * uz,
        (_C5 * 0.5) * (2.0 * uz * uz - ux * ux - uy * uy),
        _C15 * ux * uz,
        (_C15 * 0.5) * (ux * ux - uy * uy),
    ]
    sh9m = jnp.stack(sh_comps, axis=-1) * mask3[..., None]
    sh9m_flat = sh9m.reshape(P, 9)
    shfM = jnp.dot(sh9m_flat, shmap, preferred_element_type=jnp.float32)

    rbf = jnp.exp(-0.5 * jnp.square((el[..., None] - centers) / _WD)
                  ).reshape(P, _NB)

    x = jnp.dot(onehot.reshape(N, 5), atom_table,
                preferred_element_type=jnp.float32)

    z1_0, z2_0, w0 = _mlp_fwd(rbf, deg_W1, deg_b1, deg_W2, deg_b2, deg_W3)
    T0 = (w0 * shfM).reshape(B, _A, _A, _D)
    x = x + T0.sum(axis=2).reshape(N, _D) * _INV_SQ_DEG

    xs = []
    mlp_cache = []
    for b in range(_NBLK):
        xs.append(x)
        z1, z2, w = _mlp_fwd(rbf, blk_W1[b], blk_b1[b], blk_W2[b], blk_b2[b],
                             blk_W3[b])
        mlp_cache.append((z1, z2, w))
        v = jnp.dot(x, blk_Wv[b], preferred_element_type=jnp.float32)
        T4 = (w * shfM).reshape(B, _A, _A, _D)
        agg = (T4 * v.reshape(B, 1, _A, _D)).sum(axis=2).reshape(N, _D)
        x = x + jnp.dot(agg * _INV_SQ_DEG, blk_Wo[b],
                        preferred_element_type=jnp.float32)
    x5 = x

    mu = jnp.mean(x5, axis=-1, keepdims=True)
    var = jnp.mean(jnp.square(x5 - mu), axis=-1, keepdims=True)
    istd = 1.0 / jnp.sqrt(var + 1e-5)
    y = (x5 - mu) * istd
    xn = y * ln_g + ln_b
    h = jnp.dot(xn, head_W1, preferred_element_type=jnp.float32)
    hp = jnp.concatenate([_silu(h[:, :128]), h[:, 128:]], axis=1)
    node_e = jnp.sum(hp * head_W2t, axis=-1)
    energy = node_e.reshape(B, _A).sum(axis=1, keepdims=True) * _INV_SQ_NODES

    hp_bar = jnp.broadcast_to(head_W2t * _INV_SQ_NODES, (N, _D))
    h_bar = jnp.concatenate(
        [hp_bar[:, :128] * _dsilu(h[:, :128]), hp_bar[:, 128:]], axis=1)
    xn_bar = jnp.dot(h_bar, head_W1.T, preferred_element_type=jnp.float32)
    y_bar = xn_bar * ln_g
    x_bar = istd * (y_bar
                    - jnp.mean(y_bar, axis=-1, keepdims=True)
                    - y * jnp.mean(y_bar * y, axis=-1, keepdims=True))

    rbf_bar = jnp.zeros((P, _NB), jnp.float32)
    sh9m_bar = jnp.zeros((P, 9), jnp.float32)

    for b in range(_NBLK - 1, -1, -1):
        z1, z2, w = mlp_cache[b]
        v = jnp.dot(xs[b], blk_Wv[b], preferred_element_type=jnp.float32)
        agg_bar = jnp.dot(x_bar, blk_Wo[b].T,
                          preferred_element_type=jnp.float32) * _INV_SQ_DEG
        agg_bar4 = agg_bar.reshape(B, _A, 1, _D)
        T4 = (w * shfM).reshape(B, _A, _A, _D)
        v_bar = (T4 * agg_bar4).sum(axis=1).reshape(N, _D)
        T_bar = (agg_bar4 * v.reshape(B, 1, _A, _D)).reshape(P, _D)
        w_bar = T_bar * shfM
        Tw = T_bar * w
        sh9m_bar = sh9m_bar + jnp.dot(Tw, shmap.T, preferred_element_type=jnp.float32)
        rbf_bar = rbf_bar + _mlp_bwd(w_bar, z1, z2, blk_W1[b], blk_W2[b], blk_W3[b])
        x_bar = x_bar + jnp.dot(v_bar, blk_Wv[b].T, preferred_element_type=jnp.float32)

    T0_bar = jnp.broadcast_to(
        (x_bar * _INV_SQ_DEG).reshape(B, _A, 1, _D), (B, _A, _A, _D)).reshape(P, _D)
    w_bar = T0_bar * shfM
    Tw = T0_bar * w0
    sh9m_bar = sh9m_bar + jnp.dot(Tw, shmap.T, preferred_element_type=jnp.float32)
    rbf_bar = rbf_bar + _mlp_bwd(w_bar, z1_0, z2_0, deg_W1, deg_W2, deg_W3)

    rbf_bar4 = rbf_bar.reshape(B, _A, _A, _NB)
    rbf4 = rbf.reshape(B, _A, _A, _NB)
    el_bar = ((rbf_bar4 * rbf4 * (centers - el[..., None])).sum(axis=-1)
              * (1.0 / (_WD * _WD)))

    shb = sh9m_bar.reshape(B, _A, _A, 9) * mask3[..., None]
    s1 = shb[..., 1]; s2 = shb[..., 2]; s3 = shb[..., 3]
    s4 = shb[..., 4]; s5 = shb[..., 5]; s6 = shb[..., 6]
    s7 = shb[..., 7]; s8 = shb[..., 8]
    ux_bar = _C3 * s1 + _C15 * (uy * s4 + uz * s7) - _C5 * ux * s6 + _C15 * ux * s8
    uy_bar = _C3 * s2 + _C15 * (ux * s4 + uz * s5) - _C5 * uy * s6 - _C15 * uy * s8
    uz_bar = _C3 * s3 + _C15 * (uy * s5 + ux * s7) + 2.0 * _C5 * uz * s6

    evx_bar = ux_bar * inv
    evy_bar = uy_bar * inv
    evz_bar = uz_bar * inv
    inv_bar = ux_bar * evx + uy_bar * evy + uz_bar * evz
    el_bar = el_bar - inv_bar * inv * inv
    s_bar = el_bar * 0.5 * inv
    evx_bar = evx_bar + 2.0 * evx * s_bar
    evy_bar = evy_bar + 2.0 * evy * s_bar
    evz_bar = evz_bar + 2.0 * evz * s_bar

    Fx = evx_bar.sum(axis=2) - evx_bar.sum(axis=1)
    Fy = evy_bar.sum(axis=2) - evy_bar.sum(axis=1)
    Fz = evz_bar.sum(axis=2) - evz_bar.sum(axis=1)
    return energy, Fx, Fy, Fz


def _pallas_body(rt_ref, oh_ref, mask_ref, shmap_ref, cen_ref,
                 at_ref, dW1_ref, db1_ref, dW2_ref, db2_ref, dW3_ref,
                 bW1_ref, bb1_ref, bW2_ref, bb2_ref, bW3_ref, bWv_ref, bWo_ref,
                 lng_ref, lnb_ref, hW1_ref, hW2_ref,
                 e_ref, f_ref):
    energy, Fx, Fy, Fz = _block_compute(
        rt_ref[:, 0, :], rt_ref[:, 1, :], rt_ref[:, 2, :], oh_ref[...],
        mask_ref[...], shmap_ref[...], cen_ref[0],
        at_ref[...], dW1_ref[...], db1_ref[...], dW2_ref[...], db2_ref[...],
        dW3_ref[...],
        bW1_ref[...], bb1_ref[...], bW2_ref[...], bb2_ref[...], bW3_ref[...],
        bWv_ref[...], bWo_ref[...],
        lng_ref[...], lnb_ref[...], hW1_ref[...], hW2_ref[...])
    e_ref[:, 0, :] = energy
    f_ref[:, 0, :] = Fx
    f_ref[:, 1, :] = Fy
    f_ref[:, 2, :] = Fz


def kernel(node_atom, pos, batch, edge_src, edge_dst, atom_table,
           deg_W1, deg_b1, deg_W2, deg_b2, deg_W3,
           blk_W1, blk_b1, blk_W2, blk_b2, blk_W3, blk_Wv, blk_Wo,
           ln_g, ln_b, head_W1, head_W2):
    B = 2
    node_atom = node_atom.astype(jnp.int32)
    edge_src = edge_src.astype(jnp.int32)
    edge_dst = edge_dst.astype(jnp.int32)

    R3 = pos.reshape(_M, _A, 3).transpose(0, 2, 1)
    onehot = (node_atom[:, None] ==
              jnp.array([1, 6, 7, 8, 9], jnp.int32)[None, :]
              ).astype(jnp.float32).reshape(_M, _A, 5)
    mask = jnp.zeros((_M, _A, _A), jnp.float32).at[
        edge_dst // _A, edge_dst % _A, edge_src % _A].set(1.0)
    shmap = jnp.asarray(_SHMAP_NP)

    grid = (_M // B,)
    full = lambda shape: pl.BlockSpec(shape, lambda m: tuple(0 for _ in shape))
    in_specs = [
        pl.BlockSpec((B, 3, _A), lambda m: (m, 0, 0)),
        pl.BlockSpec((B, _A, 5), lambda m: (m, 0, 0)),
        pl.BlockSpec((B, _A, _A), lambda m: (m, 0, 0)),
        full((9, _D)),
        full((1, _NB)),
        full((5, _D)),
        full((_NB, 64)), full((1, 64)), full((64, 64)), full((1, 64)),
        full((64, _D)),
        full((_NBLK, _NB, 64)), full((_NBLK, 1, 64)),
        full((_NBLK, 64, 64)), full((_NBLK, 1, 64)),
        full((_NBLK, 64, _D)),
        full((_NBLK, _D, _D)), full((_NBLK, _D, _D)),
        full((1, _D)), full((1, _D)),
        full((_D, _D)), full((1, _D)),
    ]
    out_specs = [
        pl.BlockSpec((B, 1, 1), lambda m: (m, 0, 0)),
        pl.BlockSpec((B, 3, _A), lambda m: (m, 0, 0)),
    ]
    energy, fT = pl.pallas_call(
        _pallas_body,
        grid=grid,
        in_specs=in_specs,
        out_specs=out_specs,
        out_shape=[
            jax.ShapeDtypeStruct((_M, 1, 1), jnp.float32),
            jax.ShapeDtypeStruct((_M, 3, _A), jnp.float32),
        ],
        compiler_params=pltpu.CompilerParams(
            dimension_semantics=("arbitrary",)),
    )(R3, onehot, mask, shmap, jnp.asarray(_CENTERS_NP).reshape(1, _NB),
      atom_table,
      deg_W1, deg_b1.reshape(1, 64), deg_W2, deg_b2.reshape(1, 64), deg_W3,
      blk_W1, blk_b1.reshape(_NBLK, 1, 64), blk_W2, blk_b2.reshape(_NBLK, 1, 64),
      blk_W3, blk_Wv, blk_Wo,
      ln_g.reshape(1, _D), ln_b.reshape(1, _D), head_W1, head_W2.T)

    forces = fT.transpose(0, 2, 1).reshape(_M * _A, 3)
    return (energy.reshape(_M, 1), forces)
